# Initial kernel scaffold; baseline (speedup 1.0000x reference)
import jax
import jax.numpy as jnp
from jax.experimental import pallas as pl


def kernel(x, table, Wp, bp):
    raise NotImplementedError("write your pallas kernel here")


# speedup vs baseline: 1.0237x; 1.0237x over previous
import functools

import jax
import jax.numpy as jnp
from jax import lax
from jax.experimental import pallas as pl
from jax.experimental.pallas import tpu as pltpu
from jax.experimental.pallas import tpu_sc as plsc

_D = 64
_L = 20
_NC = 2
_NS = 16
_NW = _NC * _NS

_B, _H, _W = 1024, 5, 4
_N_CELLS = _B * _H * _W
_CPW = _N_CELLS // _NW
_C = 64
_NCH = _CPW // _C
_IPG = 128
_G = _C * _L // _IPG
_N_IDX = _N_CELLS * _L


def _sc_pool(idx_hbm, table_hbm, out_hbm, idx_v, rows_v, acc_v, sem):
    wid = lax.axis_index("s") * _NC + lax.axis_index("c")

    def chunk(ch, carry):
        cell_base = wid * _CPW + ch * _C
        idx_off = cell_base * _L
        pltpu.sync_copy(idx_hbm.at[pl.ds(idx_off, _C * _L)], idx_v)
        copies = []
        for j in range(_G):
            copies.append(
                pltpu.async_copy(
                    table_hbm.at[idx_v.at[pl.ds(j * _IPG, _IPG)]],
                    rows_v.at[pl.ds(j * _IPG, _IPG)],
                    sem,
                )
            )
        for cp in copies:
            cp.wait()

        def cell(c, carry2):
            r0 = c * _L
            for d in range(_D // 16):
                s = rows_v[r0, pl.ds(d * 16, 16)]
                for l in range(1, _L):
                    s = s + rows_v[r0 + l, pl.ds(d * 16, 16)]
                acc_v[c, pl.ds(d * 16, 16)] = s
            return carry2

        lax.fori_loop(0, _C, cell, 0, unroll=False)
        pltpu.sync_copy(acc_v, out_hbm.at[pl.ds(cell_base, _C)])
        return carry

    lax.fori_loop(0, _NCH, chunk, 0, unroll=False)


_sc_pool_call = functools.partial(
    pl.kernel,
    mesh=plsc.VectorSubcoreMesh(core_axis_name="c", subcore_axis_name="s"),
    out_type=jax.ShapeDtypeStruct((_N_CELLS, _D), jnp.float32),
    scratch_types=[
        pltpu.VMEM((_C * _L,), jnp.int32),
        pltpu.VMEM((_C * _L, _D), jnp.float32),
        pltpu.VMEM((_C, _D), jnp.float32),
        pltpu.SemaphoreType.DMA,
    ],
    compiler_params=pltpu.CompilerParams(use_tc_tiling_on_sc=False),
)(_sc_pool)


_SQRT1_2 = 0.7071067811865476


def _tc_head(sums_ref, wp_ref, bp_ref, o_ref):
    h = sums_ref[:] * (1.0 / _L)
    g = 0.5 * h * (1.0 + lax.erf(h * _SQRT1_2))
    w = wp_ref[0, :]
    o_ref[:] = jnp.sum(g * w[None, :], axis=1, keepdims=True) + bp_ref[0]


def kernel(x, table, Wp, bp):
    idx = x.reshape(_N_IDX).astype(jnp.int32)
    sums = _sc_pool_call(idx, table)
    out = pl.pallas_call(
        _tc_head,
        grid=(_N_CELLS // 1024,),
        in_specs=[
            pl.BlockSpec((1024, _D), lambda i: (i, 0)),
            pl.BlockSpec((1, _D), lambda i: (0, 0)),
            pl.BlockSpec(memory_space=pltpu.SMEM),
        ],
        out_specs=pl.BlockSpec((1024, 1), lambda i: (i, 0)),
        out_shape=jax.ShapeDtypeStruct((_N_CELLS, 1), jnp.float32),
    )(sums, Wp, bp)
    return out.reshape(_B, _H, _W)
